# Initial kernel scaffold; baseline (speedup 1.0000x reference)
import functools

import jax
import jax.numpy as jnp
from jax import lax
from jax.experimental import pallas as pl
from jax.experimental.pallas import tpu as pltpu
from jax.experimental.pallas import tpu_sc as plsc

NUM_EMB = 1_000_000
DIM = 64
B_TOK = 16384
SEQ = 50
B_FLAT = B_TOK * SEQ
NC, NS = 2, 16
NW = NC * NS
PER_W = B_FLAT // NW
CHUNK = 128
NSTEP = PER_W // CHUNK


def _sc_gather(idx, emb):
    mesh = plsc.VectorSubcoreMesh(core_axis_name="c", subcore_axis_name="s")

    @functools.partial(
        pl.kernel,
        mesh=mesh,
        out_type=jax.ShapeDtypeStruct((NW, NSTEP, CHUNK, DIM), jnp.float32),
        scratch_types=[
            pltpu.VMEM((NSTEP, CHUNK), jnp.int32),
            pltpu.VMEM((2, CHUNK, DIM), jnp.float32),
            pltpu.SemaphoreType.DMA,
            pltpu.SemaphoreType.DMA,
        ],
    )
    def k(idx_hbm, table_hbm, out_hbm, idx_v, rows_v, gsem, osem):
        wid = lax.axis_index("s") * NC + lax.axis_index("c")
        pltpu.sync_copy(idx_hbm.at[wid], idx_v)

        def body(j, _):
            pltpu.async_copy(table_hbm.at[idx_v.at[j]], rows_v.at[0], gsem).wait()
            pltpu.async_copy(rows_v.at[0], out_hbm.at[wid, j], osem).wait()
            return 0

        lax.fori_loop(0, NSTEP, body, 0)

    return k(idx, emb)


def kernel(token_ids, emb):
    idx = token_ids.reshape(NW, NSTEP, CHUNK).astype(jnp.int32)
    out = _sc_gather(idx, emb)
    return out.reshape(B_TOK, SEQ, DIM)


# speedup vs baseline: 1.6874x; 1.6874x over previous
import functools

import jax
import jax.numpy as jnp
from jax import lax
from jax.experimental import pallas as pl
from jax.experimental.pallas import tpu as pltpu
from jax.experimental.pallas import tpu_sc as plsc

NUM_EMB = 1_000_000
DIM = 64
B_TOK = 16384
SEQ = 50
B_FLAT = B_TOK * SEQ
NC, NS = 2, 16
NW = NC * NS
PER_W = B_FLAT // NW
CHUNK = 128
NSTEP = PER_W // CHUNK


def _sc_gather(idx, emb):
    mesh = plsc.VectorSubcoreMesh(core_axis_name="c", subcore_axis_name="s")

    @functools.partial(
        pl.kernel,
        mesh=mesh,
        out_type=jax.ShapeDtypeStruct((NW, NSTEP, CHUNK, DIM), jnp.float32),
        scratch_types=[
            pltpu.VMEM((NSTEP, CHUNK), jnp.int32),
            pltpu.VMEM((2, CHUNK, DIM), jnp.float32),
            pltpu.SemaphoreType.DMA,
            pltpu.SemaphoreType.DMA,
        ],
        compiler_params=pltpu.CompilerParams(use_tc_tiling_on_sc=False),
    )
    def k(idx_hbm, table_hbm, out_hbm, idx_v, rows_v, gsem, osem):
        wid = lax.axis_index("s") * NC + lax.axis_index("c")
        pltpu.sync_copy(idx_hbm.at[wid], idx_v)

        def body(j, _):
            pltpu.async_copy(table_hbm.at[idx_v.at[j]], rows_v.at[0], gsem).wait()
            pltpu.async_copy(rows_v.at[0], out_hbm.at[wid, j], osem).wait()
            return 0

        lax.fori_loop(0, NSTEP, body, 0)

    return k(idx, emb)


def kernel(token_ids, emb):
    idx = token_ids.reshape(NW, NSTEP, CHUNK).astype(jnp.int32)
    out = _sc_gather(idx, emb)
    return out.reshape(B_TOK, SEQ, DIM)


# speedup vs baseline: 1.8728x; 1.1099x over previous
import functools

import jax
import jax.numpy as jnp
from jax import lax
from jax.experimental import pallas as pl
from jax.experimental.pallas import tpu as pltpu
from jax.experimental.pallas import tpu_sc as plsc

NUM_EMB = 1_000_000
DIM = 64
B_TOK = 16384
SEQ = 50
B_FLAT = B_TOK * SEQ
NC, NS = 2, 16
NW = NC * NS
PER_W = B_FLAT // NW
CHUNK = 128
NSTEP = PER_W // CHUNK
K = 5
NGROUP = NSTEP // K
NPAIR = NGROUP // 2


def _sc_gather(idx, emb):
    mesh = plsc.VectorSubcoreMesh(core_axis_name="c", subcore_axis_name="s")

    @functools.partial(
        pl.kernel,
        mesh=mesh,
        out_type=jax.ShapeDtypeStruct((NW, NSTEP, CHUNK, DIM), jnp.float32),
        scratch_types=[
            pltpu.VMEM((NSTEP, CHUNK), jnp.int32),
            pltpu.VMEM((2, K, CHUNK, DIM), jnp.float32),
            pltpu.SemaphoreType.DMA,
            pltpu.SemaphoreType.DMA,
            pltpu.SemaphoreType.DMA,
            pltpu.SemaphoreType.DMA,
        ],
        compiler_params=pltpu.CompilerParams(use_tc_tiling_on_sc=False),
    )
    def k(idx_hbm, table_hbm, out_hbm, idx_v, rows_v, gsem_a, gsem_b,
          osem_a, osem_b):
        wid = lax.axis_index("s") * NC + lax.axis_index("c")
        pltpu.sync_copy(idx_hbm.at[wid], idx_v)

        def fire_gathers(g, s, sem):
            for b in range(K):
                pltpu.async_copy(
                    table_hbm.at[idx_v.at[g * K + b]], rows_v.at[s, b], sem)

        def drain_gathers(s, sem):
            for b in range(K):
                pltpu.make_async_copy(
                    out_hbm.at[0, 0], rows_v.at[s, b], sem).wait()

        def fire_writes(g, s, sem):
            for b in range(K):
                pltpu.async_copy(
                    rows_v.at[s, b], out_hbm.at[wid, g * K + b], sem)

        def wait_writes(s, sem):
            for b in range(K):
                pltpu.make_async_copy(
                    rows_v.at[s, b], out_hbm.at[0, b], sem).wait()

        fire_gathers(0, 0, gsem_a)
        fire_gathers(1, 1, gsem_b)
        drain_gathers(0, gsem_a)
        fire_writes(0, 0, osem_a)
        wait_writes(0, osem_a)
        fire_gathers(2, 0, gsem_a)
        drain_gathers(1, gsem_b)
        fire_writes(1, 1, osem_b)

        def body(t, _):
            wait_writes(1, osem_b)
            fire_gathers(2 * t + 1, 1, gsem_b)
            drain_gathers(0, gsem_a)
            fire_writes(2 * t, 0, osem_a)
            wait_writes(0, osem_a)
            fire_gathers(2 * t + 2, 0, gsem_a)
            drain_gathers(1, gsem_b)
            fire_writes(2 * t + 1, 1, osem_b)
            return 0

        lax.fori_loop(1, NPAIR - 1, body, 0)

        wait_writes(1, osem_b)
        fire_gathers(NGROUP - 1, 1, gsem_b)
        drain_gathers(0, gsem_a)
        fire_writes(NGROUP - 2, 0, osem_a)
        drain_gathers(1, gsem_b)
        fire_writes(NGROUP - 1, 1, osem_b)
        wait_writes(0, osem_a)
        wait_writes(1, osem_b)

    return k(idx, emb)


def kernel(token_ids, emb):
    idx = token_ids.reshape(NW, NSTEP, CHUNK).astype(jnp.int32)
    out = _sc_gather(idx, emb)
    return out.reshape(B_TOK, SEQ, DIM)
